# baseline (speedup 1.0000x reference)
import functools

import jax
import jax.numpy as jnp
from jax.experimental import pallas as pl
from jax.experimental.pallas import tpu as pltpu

_TARGET_BLOCK_BYTES = 4 * 1024 * 1024


def _largest_divisor_at_most(n, cap):
    cap = max(1, min(n, cap))
    for t in range(cap, 0, -1):
        if n % t == 0:
            return t
    return 1


def _rowmean_kernel(x_ref, o_ref, *, inv_s):
    s = jnp.sum(x_ref[...], axis=-1, keepdims=True, dtype=jnp.float32)
    o_ref[...] = (s * inv_s).astype(o_ref.dtype)


def kernel(x):
    n, c, d, h, w = x.shape
    s = d * h * w
    rows = n * c
    x2 = x.reshape(rows, s)
    itemsize = x2.dtype.itemsize

    row_bytes = s * itemsize
    rb_cap = max(8, _TARGET_BLOCK_BYTES // row_bytes)
    if rows >= 16:
        rb_cap = min(rb_cap, rows // 2)
    rb = _largest_divisor_at_most(rows // 8, rb_cap // 8) * 8 \
        if rows % 8 == 0 else _largest_divisor_at_most(rows, rb_cap)
    nb = rows // rb

    cost = pl.CostEstimate(
        flops=rows * s,
        transcendentals=0,
        bytes_accessed=rows * s * itemsize + rows * itemsize,
    )

    out = pl.pallas_call(
        functools.partial(_rowmean_kernel, inv_s=1.0 / s),
        out_shape=jax.ShapeDtypeStruct((rows, 1), x2.dtype),
        grid_spec=pltpu.PrefetchScalarGridSpec(
            num_scalar_prefetch=0,
            grid=(nb,),
            in_specs=[pl.BlockSpec((rb, s), lambda i: (i, 0))],
            out_specs=pl.BlockSpec((rb, 1), lambda i: (i, 0)),
        ),
        compiler_params=pltpu.CompilerParams(
            dimension_semantics=("parallel",),
        ),
        cost_estimate=cost,
    )(x2)

    return out.reshape(-1, 512)


# speedup vs baseline: 1.5115x; 1.5115x over previous
import functools

import jax
import jax.numpy as jnp
from jax.experimental import pallas as pl
from jax.experimental.pallas import tpu as pltpu

_TARGET_BLOCK_BYTES = 4 * 1024 * 1024


def _largest_divisor_at_most(n, cap):
    cap = max(1, min(n, cap))
    for t in range(cap, 0, -1):
        if n % t == 0:
            return t
    return 1


def _rowmean_kernel(x_ref, o_ref, *, inv_s):
    s = jnp.sum(x_ref[...], axis=-1, keepdims=True, dtype=jnp.float32)
    o_ref[...] = (s * inv_s).astype(o_ref.dtype)


def kernel(x):
    n, c, d, h, w = x.shape
    s = d * h * w
    x3 = x.reshape(n, c, s)
    itemsize = x3.dtype.itemsize

    per_sample = c * s * itemsize
    tb_cap = max(1, _TARGET_BLOCK_BYTES // per_sample)
    if n >= 2:
        tb_cap = min(tb_cap, n // 2)
    tb = _largest_divisor_at_most(n, tb_cap)
    nb = n // tb

    cost = pl.CostEstimate(
        flops=n * c * s,
        transcendentals=0,
        bytes_accessed=n * c * s * itemsize + n * c * itemsize,
    )

    out = pl.pallas_call(
        functools.partial(_rowmean_kernel, inv_s=1.0 / s),
        out_shape=jax.ShapeDtypeStruct((n, c, 1), x3.dtype),
        grid_spec=pltpu.PrefetchScalarGridSpec(
            num_scalar_prefetch=0,
            grid=(nb,),
            in_specs=[pl.BlockSpec((tb, c, s), lambda i: (i, 0, 0))],
            out_specs=pl.BlockSpec((tb, c, 1), lambda i: (i, 0, 0)),
        ),
        compiler_params=pltpu.CompilerParams(
            dimension_semantics=("parallel",),
        ),
        cost_estimate=cost,
    )(x3)

    return out.reshape(-1, 512)
